# baseline (speedup 1.0000x reference)
import functools

import jax
import jax.numpy as jnp
from jax import lax
from jax.experimental import pallas as pl
from jax.experimental.pallas import tpu as pltpu
from jax.experimental.pallas import tpu_sc as plsc
from jax.experimental import layout as jax_layout

NC = 2
NS = 16
NW = NC * NS

CB = 4
NBUF = 4


def _kernel_impl(qus, table):
    batch, seq = qus.shape
    vocab, dim = table.shape
    assert batch % (NW * CB) == 0
    b_per_w = batch // NW
    n_chunks = b_per_w // CB

    idx_in = qus.astype(jnp.int32).reshape(NW, batch // (NW * CB), CB * seq)

    mesh = plsc.VectorSubcoreMesh(core_axis_name="c", subcore_axis_name="s")
    LAG = NBUF - 1

    @functools.partial(
        pl.kernel,
        out_type=jax.ShapeDtypeStruct((batch, seq, dim), jnp.float32),
        mesh=mesh,
        scratch_types=[
            pltpu.VMEM((n_chunks, CB * seq), jnp.int32),
            pltpu.VMEM((NBUF, CB * seq, dim), jnp.float32),
            [pltpu.SemaphoreType.DMA] * NBUF,
            [pltpu.SemaphoreType.DMA] * NBUF,
        ],
        compiler_params=pltpu.CompilerParams(
            use_tc_tiling_on_sc=True, needs_layout_passes=True
        ),
    )
    def emb(idx_hbm, table_hbm, out_hbm, idx_v, rows_v, gsems, wsems):
        wid = lax.axis_index("s") * NC + lax.axis_index("c")
        base_b = pl.multiple_of(wid * b_per_w, b_per_w)
        pltpu.sync_copy(idx_hbm.at[wid], idx_v)
        gd = [None] * NBUF
        wd = [None] * NBUF
        for j in range(n_chunks + LAG):
            if j < n_chunks:
                b = j % NBUF
                if wd[b] is not None:
                    wd[b].wait()
                    wd[b] = None
                gd[b] = pltpu.async_copy(
                    table_hbm.at[idx_v.at[j]],
                    rows_v.at[b],
                    gsems[b],
                )
            k = j - LAG
            if k >= 0:
                bk = k % NBUF
                gd[bk].wait()
                b0 = pl.multiple_of(base_b + k * CB, CB)
                wd[bk] = pltpu.async_copy(
                    rows_v.at[bk].reshape(CB, seq, dim),
                    out_hbm.at[pl.ds(b0, CB)],
                    wsems[bk],
                )
        for b in range(NBUF):
            if wd[b] is not None:
                wd[b].wait()

    return emb(idx_in, table)


kernel = jax.jit(_kernel_impl)
